# baseline (speedup 1.0000x reference)
import jax
import jax.numpy as jnp
from jax.experimental import pallas as pl
from jax.experimental.pallas import tpu as pltpu

_HP1, _WP1 = 30, 30
_HP2, _WP2 = 16, 16
_HP3, _WP3 = 9, 9
_R1 = _HP1 * _WP1
_R2 = _HP2 * _WP2
_R3 = _HP3 * _WP3

_B = 32


def _conv3x3(a, wp, w_ref):
    rtot = a.shape[0]
    g = wp + 1
    ae = jnp.pad(a, ((g, g), (0, 0)))
    acc = None
    for dy in range(3):
        for dx in range(3):
            s = (dy - 1) * wp + (dx - 1)
            tap = ae[g + s: g + s + rtot, :]
            part = jnp.dot(tap, w_ref[dy * 3 + dx],
                           preferred_element_type=jnp.float32)
            acc = part if acc is None else acc + part
    return acc


def _pool_max(c, wp, k):
    rtot = c.shape[0]
    smax = (k - 1) * (wp + 1)
    ce = jnp.pad(c, ((0, smax), (0, 0)))
    lcol = rtot + (k - 1) * wp
    m = ce[0:lcol, :]
    for j in range(1, k):
        m = jnp.maximum(m, ce[j:j + lcol, :])
    r = m[0:rtot, :]
    for i in range(1, k):
        r = jnp.maximum(r, m[i * wp: i * wp + rtot, :])
    return r


def _sel_odd_rows(m, b, hp, wp, c):
    q = m.reshape(b * hp * wp // 2, 2, c)[:, 1:2, :]
    q = q.reshape(b * hp // 2, wp, c)[:, wp // 2:, :]
    return q.reshape(b, hp // 2, wp // 2, c)


def _embed(sel, b, nq, wpo, c):
    zw = jnp.zeros((b, nq, 1, c), sel.dtype)
    row = jnp.concatenate([zw, sel, zw], axis=2)
    zh = jnp.zeros((b, 1, wpo, c), sel.dtype)
    out = jnp.concatenate([zh, row, zh], axis=1)
    return out.reshape(b * (nq + 2) * wpo, c)


def _pool_stage(cf, b, hp, wp, k, nq, wpo, bias_ref):
    m = _pool_max(cf, wp, k)
    sel = _sel_odd_rows(m, b, hp, wp, cf.shape[1])[:, :nq, :nq, :]
    act = jnp.maximum(sel + bias_ref[0], 0.0).astype(jnp.bfloat16)
    return _embed(act, b, nq, wpo, cf.shape[1])


def _ring_stage(cf, b, hp, wp, bias_ref):
    c = cf.shape[1]
    x4 = cf.reshape(b, hp, wp, c)
    inner = (x4[:, 1:hp - 1, 1:wp - 1, :] + bias_ref[0]).astype(jnp.bfloat16)
    return _embed(inner, b, hp - 2, wp, c)


def _net_kernel(x1_ref, w1_ref, b1_ref, w2_ref, b2_ref, w3_ref, b3_ref,
                w4_ref, b4_ref, w5_ref, b5_ref,
                f1_ref, fb1_ref, f2_ref, fb2_ref, f3_ref, fb3_ref, o_ref):
    b = _B
    x1 = x1_ref[...].reshape(b * _R1, 9)
    c1 = jnp.dot(x1, w1_ref[...], preferred_element_type=jnp.float32)
    a2 = _pool_stage(c1, b, _HP1, _WP1, 2, 14, _WP2, b1_ref)

    c2 = _conv3x3(a2, _WP2, w2_ref)
    a3 = _pool_stage(c2, b, _HP2, _WP2, 2, 7, _WP3, b2_ref)

    c3 = _conv3x3(a3, _WP3, w3_ref)
    a4 = _ring_stage(c3, b, _HP3, _WP3, b3_ref)
    c4 = _conv3x3(a4, _WP3, w4_ref)
    a5 = _ring_stage(c4, b, _HP3, _WP3, b4_ref)

    c5 = _conv3x3(a5, _WP3, w5_ref)
    m5 = _pool_max(c5, _WP3, 3).reshape(b, _HP3, _WP3, 256)
    q = m5[:, :, 1:7, :].reshape(b * _HP3 * 3, 2, 256)[:, 0:1, :]
    q = q.reshape(b, _HP3, 3, 256)[:, 1:7, :, :]
    sel = q.reshape(b * 3, 6, 256)[:, 0:3, :].reshape(b, 3, 3, 256)
    afc = jnp.maximum(sel + b5_ref[0], 0.0).astype(jnp.bfloat16)

    h = fb1_ref[...]
    for i in range(3):
        for j in range(3):
            v = afc[:, i:i + 1, j:j + 1, :].reshape(b, 256)
            h = h + jnp.dot(v, f1_ref[i * 3 + j],
                            preferred_element_type=jnp.float32)
    h = jnp.dot(h.astype(jnp.bfloat16), f2_ref[...],
                preferred_element_type=jnp.float32) + fb2_ref[...]
    out = jnp.dot(h.astype(jnp.bfloat16), f3_ref[...],
                  preferred_element_type=jnp.float32) + fb3_ref[...]
    o_ref[...] = out


def _conv1_taps(x_nchw):
    n = x_nchw.shape[0]
    x = x_nchw[:, 0, :, :].astype(jnp.float32)
    xp = jnp.pad(x, ((0, 0), (2, 2), (2, 2)))
    taps = [xp[:, dy:dy + _HP1, dx:dx + _WP1]
            for dy in range(3) for dx in range(3)]
    t = jnp.stack(taps, axis=-1)
    return t.reshape(n, _R1, 9).astype(jnp.bfloat16)


def _const_spec(a):
    nd = a.ndim
    return pl.BlockSpec(a.shape, lambda *_: (0,) * nd)


@jax.jit
def _forward(x, w1, b1, w2, b2, w3, b3, w4, b4, w5, b5,
             f1, fb1, f2, fb2, f3, fb3):
    n = x.shape[0]
    x1 = _conv1_taps(x)
    params = (w1, b1, w2, b2, w3, b3, w4, b4, w5, b5,
              f1, fb1, f2, fb2, f3, fb3)

    in_specs = [pl.BlockSpec((_B, _R1, 9), lambda g: (g, 0, 0))]
    in_specs += [_const_spec(p) for p in params]

    cost = pl.CostEstimate(
        flops=172_000_000 * n,
        transcendentals=0,
        bytes_accessed=9_000_000 + 2 * x1.size,
    )

    out = pl.pallas_call(
        _net_kernel,
        out_shape=jax.ShapeDtypeStruct((n, 10), jnp.float32),
        grid_spec=pltpu.PrefetchScalarGridSpec(
            num_scalar_prefetch=0,
            grid=(n // _B,),
            in_specs=in_specs,
            out_specs=pl.BlockSpec((_B, 10), lambda g: (g, 0)),
        ),
        compiler_params=pltpu.CompilerParams(
            dimension_semantics=("parallel",),
            vmem_limit_bytes=96 * 1024 * 1024,
        ),
        cost_estimate=cost,
    )(x1, *params)
    return out


def kernel(x, w1, b1, s1, w2, b2, s2, w3, b3, r34, w4, b4, w5, b5, s5,
           f1, fb1, f2, fb2, f3, fb3):
    del s1, s2, r34, s5
    return _forward(x, w1, b1, w2, b2, w3, b3, w4, b4, w5, b5,
                    f1, fb1, f2, fb2, f3, fb3)


# speedup vs baseline: 1.0350x; 1.0350x over previous
import jax
import jax.numpy as jnp
from jax.experimental import pallas as pl
from jax.experimental.pallas import tpu as pltpu

_HP1, _WP1 = 30, 30
_HP2, _WP2 = 16, 16
_HP3, _WP3 = 9, 9
_R1 = _HP1 * _WP1
_R2 = _HP2 * _WP2
_R3 = _HP3 * _WP3

_B = 32


def _conv3x3(a, wp, w_ref):
    rtot = a.shape[0]
    g = wp + 1
    ae = jnp.pad(a, ((g, g), (0, 0)))
    acc = None
    for dy in range(3):
        for dx in range(3):
            s = (dy - 1) * wp + (dx - 1)
            tap = ae[g + s: g + s + rtot, :].astype(jnp.bfloat16)
            part = jnp.dot(tap, w_ref[dy * 3 + dx],
                           preferred_element_type=jnp.float32)
            acc = part if acc is None else acc + part
    return acc


def _pool_max(c, wp, k):
    rtot = c.shape[0]
    smax = (k - 1) * (wp + 1)
    ce = jnp.pad(c, ((0, smax), (0, 0)))
    lcol = rtot + (k - 1) * wp
    m = ce[0:lcol, :]
    for j in range(1, k):
        m = jnp.maximum(m, ce[j:j + lcol, :])
    r = m[0:rtot, :]
    for i in range(1, k):
        r = jnp.maximum(r, m[i * wp: i * wp + rtot, :])
    return r


def _sel_odd_rows(m, b, hp, wp, c):
    q = m.reshape(b * hp * wp // 2, 2, c)[:, 1:2, :]
    q = q.reshape(b * hp // 2, wp, c)[:, wp // 2:, :]
    return q.reshape(b, hp // 2, wp // 2, c)


def _embed(sel, b, nq, wpo, c):
    zw = jnp.zeros((b, nq, 1, c), sel.dtype)
    row = jnp.concatenate([zw, sel, zw], axis=2)
    zh = jnp.zeros((b, 1, wpo, c), sel.dtype)
    out = jnp.concatenate([zh, row, zh], axis=1)
    return out.reshape(b * (nq + 2) * wpo, c)


def _pool_stage(cf, b, hp, wp, k, nq, wpo, bias_ref):
    m = _pool_max(cf, wp, k)
    sel = _sel_odd_rows(m, b, hp, wp, cf.shape[1])[:, :nq, :nq, :]
    act = jnp.maximum(sel + bias_ref[0], 0.0)
    return _embed(act, b, nq, wpo, cf.shape[1])


def _ring_stage(cf, b, hp, wp, bias_ref):
    c = cf.shape[1]
    x4 = cf.reshape(b, hp, wp, c)
    inner = x4[:, 1:hp - 1, 1:wp - 1, :] + bias_ref[0]
    return _embed(inner, b, hp - 2, wp, c)


def _net_kernel(x1_ref, w1_ref, b1_ref, w2_ref, b2_ref, w3_ref, b3_ref,
                w4_ref, b4_ref, w5_ref, b5_ref,
                f1_ref, fb1_ref, f2_ref, fb2_ref, f3_ref, fb3_ref, o_ref):
    b = _B
    x1 = x1_ref[...].reshape(b * _R1, 9)
    c1 = jnp.dot(x1, w1_ref[...], preferred_element_type=jnp.float32)
    a2 = _pool_stage(c1, b, _HP1, _WP1, 2, 14, _WP2, b1_ref)

    c2 = _conv3x3(a2, _WP2, w2_ref)
    a3 = _pool_stage(c2, b, _HP2, _WP2, 2, 7, _WP3, b2_ref)

    c3 = _conv3x3(a3, _WP3, w3_ref)
    a4 = _ring_stage(c3, b, _HP3, _WP3, b3_ref)
    c4 = _conv3x3(a4, _WP3, w4_ref)
    a5 = _ring_stage(c4, b, _HP3, _WP3, b4_ref)

    c5 = _conv3x3(a5, _WP3, w5_ref)
    m5 = _pool_max(c5, _WP3, 3).reshape(b, _HP3, _WP3, 256)
    q = m5[:, :, 1:7, :].reshape(b * _HP3 * 3, 2, 256)[:, 0:1, :]
    q = q.reshape(b, _HP3, 3, 256)[:, 1:7, :, :]
    sel = q.reshape(b * 3, 6, 256)[:, 0:3, :].reshape(b, 3, 3, 256)
    afc = jnp.maximum(sel + b5_ref[0], 0.0)

    h = fb1_ref[...]
    for i in range(3):
        for j in range(3):
            v = afc[:, i:i + 1, j:j + 1, :].reshape(b, 256).astype(jnp.bfloat16)
            h = h + jnp.dot(v, f1_ref[i * 3 + j],
                            preferred_element_type=jnp.float32)
    h = jnp.dot(h.astype(jnp.bfloat16), f2_ref[...],
                preferred_element_type=jnp.float32) + fb2_ref[...]
    out = jnp.dot(h.astype(jnp.bfloat16), f3_ref[...],
                  preferred_element_type=jnp.float32) + fb3_ref[...]
    o_ref[...] = out


def _conv1_taps(x_nchw):
    n = x_nchw.shape[0]
    x = x_nchw[:, 0, :, :].astype(jnp.float32)
    xp = jnp.pad(x, ((0, 0), (2, 2), (2, 2)))
    taps = [xp[:, dy:dy + _HP1, dx:dx + _WP1]
            for dy in range(3) for dx in range(3)]
    t = jnp.stack(taps, axis=-1)
    return t.reshape(n, _R1, 9).astype(jnp.bfloat16)


def _const_spec(a):
    nd = a.ndim
    return pl.BlockSpec(a.shape, lambda *_: (0,) * nd)


@jax.jit
def _forward(x, w1, b1, w2, b2, w3, b3, w4, b4, w5, b5,
             f1, fb1, f2, fb2, f3, fb3):
    n = x.shape[0]
    x1 = _conv1_taps(x)
    params = (w1, b1, w2, b2, w3, b3, w4, b4, w5, b5,
              f1, fb1, f2, fb2, f3, fb3)

    in_specs = [pl.BlockSpec((_B, _R1, 9), lambda g: (g, 0, 0))]
    in_specs += [_const_spec(p) for p in params]

    cost = pl.CostEstimate(
        flops=172_000_000 * n,
        transcendentals=0,
        bytes_accessed=9_000_000 + 2 * x1.size,
    )

    out = pl.pallas_call(
        _net_kernel,
        out_shape=jax.ShapeDtypeStruct((n, 10), jnp.float32),
        grid_spec=pltpu.PrefetchScalarGridSpec(
            num_scalar_prefetch=0,
            grid=(n // _B,),
            in_specs=in_specs,
            out_specs=pl.BlockSpec((_B, 10), lambda g: (g, 0)),
        ),
        compiler_params=pltpu.CompilerParams(
            dimension_semantics=("parallel",),
            vmem_limit_bytes=96 * 1024 * 1024,
        ),
        cost_estimate=cost,
    )(x1, *params)
    return out


def kernel(x, w1, b1, s1, w2, b2, s2, w3, b3, r34, w4, b4, w5, b5, s5,
           f1, fb1, f2, fb2, f3, fb3):
    del s1, s2, r34, s5
    return _forward(x, w1, b1, w2, b2, w3, b3, w4, b4, w5, b5,
                    f1, fb1, f2, fb2, f3, fb3)


# speedup vs baseline: 1.2794x; 1.2361x over previous
import jax
import jax.numpy as jnp
from jax.experimental import pallas as pl
from jax.experimental.pallas import tpu as pltpu

_HP1, _WP1 = 30, 30
_HP2, _WP2 = 16, 16
_HP3, _WP3 = 9, 9
_R1 = _HP1 * _WP1
_R2 = _HP2 * _WP2
_R3 = _HP3 * _WP3

_B = 32


def _conv3x3(a, wp, w_ref, b_ref):
    rtot = a.shape[0]
    g = wp + 1
    ae = jnp.pad(a, ((g, g), (0, 0)))
    acc = None
    for dy in range(3):
        for dx in range(3):
            s = (dy - 1) * wp + (dx - 1)
            tap = ae[g + s: g + s + rtot, :].astype(jnp.bfloat16)
            part = jnp.dot(tap, w_ref[dy * 3 + dx],
                           preferred_element_type=jnp.float32)
            acc = part if acc is None else acc + part
    return acc + b_ref[...]


def _pool_relu(c, wp, k):
    rtot = c.shape[0]
    smax = (k - 1) * (wp + 1)
    ce = jnp.pad(c, ((0, smax), (0, 0)))
    m = c
    for i in range(k):
        for j in range(k):
            if i == 0 and j == 0:
                continue
            s = i * wp + j
            m = jnp.maximum(m, ce[s:s + rtot, :])
    return jnp.maximum(m, 0.0)


def _sel_odd_rows(m, b, hp, wp, c):
    q = m.reshape(b * hp * wp // 2, 2, c)[:, 1:2, :]
    q = q.reshape(b * hp // 2, wp, c)[:, wp // 2:, :]
    return q.reshape(b, hp // 2, wp // 2, c)


def _embed(sel, b, nq, wpo, c):
    zw = jnp.zeros((b, nq, 1, c), sel.dtype)
    row = jnp.concatenate([zw, sel, zw], axis=2)
    zh = jnp.zeros((b, 1, wpo, c), sel.dtype)
    out = jnp.concatenate([zh, row, zh], axis=1)
    return out.reshape(b * (nq + 2) * wpo, c)


def _route_pool(m, b, hp, wp, nq, hpo, wpo, c):
    sel = _sel_odd_rows(m, b, hp, wp, c)[:, :nq, :nq, :]
    return _embed(sel, b, nq, wpo, c)


def _zero_ring(cf, b, hp, wp, c):
    x4 = cf.reshape(b, hp, wp, c)
    inner = x4[:, 1:hp - 1, 1:wp - 1, :]
    zw = jnp.zeros((b, hp - 2, 1, c), cf.dtype)
    row = jnp.concatenate([zw, inner, zw], axis=2)
    zh = jnp.zeros((b, 1, wp, c), cf.dtype)
    return jnp.concatenate([zh, row, zh], axis=1).reshape(b * hp * wp, c)


def _net_kernel(x1_ref, w1_ref, b1_ref, w2_ref, b2_ref, w3_ref, b3_ref,
                w4_ref, b4_ref, w5_ref, b5_ref,
                f1_ref, fb1_ref, f2_ref, fb2_ref, f3_ref, fb3_ref, o_ref):
    b = _B
    x1 = x1_ref[...].reshape(b * _R1, 9)
    c1 = jnp.dot(x1, w1_ref[...], preferred_element_type=jnp.float32)
    c1 = c1 + b1_ref[...]
    a2 = _route_pool(_pool_relu(c1, _WP1, 2), b, _HP1, _WP1, 14,
                     _HP2, _WP2, 32)

    c2 = _conv3x3(a2, _WP2, w2_ref, b2_ref)
    a3 = _route_pool(_pool_relu(c2, _WP2, 2), b, _HP2, _WP2, 7,
                     _HP3, _WP3, 64)

    c3 = _conv3x3(a3, _WP3, w3_ref, b3_ref)
    a4 = _zero_ring(c3, b, _HP3, _WP3, 128)
    c4 = _conv3x3(a4, _WP3, w4_ref, b4_ref)
    a5 = _zero_ring(c4, b, _HP3, _WP3, 256)

    c5 = _conv3x3(a5, _WP3, w5_ref, b5_ref)
    m5 = _pool_relu(c5, _WP3, 3).reshape(b, _HP3, _WP3, 256)
    q = m5[:, :, 1:7, :].reshape(b * _HP3 * 3, 2, 256)[:, 0:1, :]
    q = q.reshape(b, _HP3, 3, 256)[:, 1:7, :, :]
    afc = q.reshape(b * 3, 6, 256)[:, 0:3, :].reshape(b, 3, 3, 256)

    h = fb1_ref[...]
    for i in range(3):
        for j in range(3):
            v = afc[:, i:i + 1, j:j + 1, :].reshape(b, 256).astype(jnp.bfloat16)
            h = h + jnp.dot(v, f1_ref[i * 3 + j],
                            preferred_element_type=jnp.float32)
    h = jnp.dot(h.astype(jnp.bfloat16), f2_ref[...],
                preferred_element_type=jnp.float32) + fb2_ref[...]
    out = jnp.dot(h.astype(jnp.bfloat16), f3_ref[...],
                  preferred_element_type=jnp.float32) + fb3_ref[...]
    o_ref[...] = out


def _conv1_taps(x_nchw):
    n = x_nchw.shape[0]
    x = x_nchw[:, 0, :, :].astype(jnp.float32)
    xp = jnp.pad(x, ((0, 0), (2, 2), (2, 2)))
    taps = [xp[:, dy:dy + _HP1, dx:dx + _WP1]
            for dy in range(3) for dx in range(3)]
    t = jnp.stack(taps, axis=-1)
    return t.reshape(n, _R1, 9).astype(jnp.bfloat16)


def _const_spec(a):
    nd = a.ndim
    return pl.BlockSpec(a.shape, lambda *_: (0,) * nd)


@jax.jit
def _forward(x, w1, b1, w2, b2, w3, b3, w4, b4, w5, b5,
             f1, fb1, f2, fb2, f3, fb3):
    n = x.shape[0]
    x1 = _conv1_taps(x)
    params = (w1, b1, w2, b2, w3, b3, w4, b4, w5, b5,
              f1, fb1, f2, fb2, f3, fb3)

    in_specs = [pl.BlockSpec((_B, _R1, 9), lambda g: (g, 0, 0))]
    in_specs += [_const_spec(p) for p in params]

    cost = pl.CostEstimate(
        flops=172_000_000 * n,
        transcendentals=0,
        bytes_accessed=9_000_000 + 2 * x1.size,
    )

    out = pl.pallas_call(
        _net_kernel,
        out_shape=jax.ShapeDtypeStruct((n, 10), jnp.float32),
        grid_spec=pltpu.PrefetchScalarGridSpec(
            num_scalar_prefetch=0,
            grid=(n // _B,),
            in_specs=in_specs,
            out_specs=pl.BlockSpec((_B, 10), lambda g: (g, 0)),
        ),
        compiler_params=pltpu.CompilerParams(
            dimension_semantics=("parallel",),
            vmem_limit_bytes=96 * 1024 * 1024,
        ),
        cost_estimate=cost,
    )(x1, *params)
    return out


def kernel(x, w1, b1, s1, w2, b2, s2, w3, b3, r34, w4, b4, w5, b5, s5,
           f1, fb1, f2, fb2, f3, fb3):
    del s1, s2, r34, s5
    return _forward(x, w1, b1, w2, b2, w3, b3, w4, b4, w5, b5,
                    f1, fb1, f2, fb2, f3, fb3)


# speedup vs baseline: 1.2958x; 1.0128x over previous
import jax
import jax.numpy as jnp
from jax.experimental import pallas as pl
from jax.experimental.pallas import tpu as pltpu

_HP1, _WP1 = 30, 30
_HP2, _WP2 = 16, 16
_HP3, _WP3 = 9, 9
_R1 = _HP1 * _WP1
_R2 = _HP2 * _WP2
_R3 = _HP3 * _WP3

_B = 32


def _conv3x3(a, wp, w_ref, b_ref):
    rtot = a.shape[0]
    g = wp + 1
    ae = jnp.pad(a, ((g, g), (0, 0)))
    acc = None
    for dy in range(3):
        for dx in range(3):
            s = (dy - 1) * wp + (dx - 1)
            tap = ae[g + s: g + s + rtot, :].astype(jnp.bfloat16)
            part = jnp.dot(tap, w_ref[dy * 3 + dx],
                           preferred_element_type=jnp.float32)
            acc = part if acc is None else acc + part
    return acc + b_ref[...]


def _pool_relu(c, wp, k):
    rtot = c.shape[0]
    smax = (k - 1) * (wp + 1)
    ce = jnp.pad(c, ((0, smax), (0, 0)))
    lcol = rtot + (k - 1) * wp
    m = ce[0:lcol, :]
    for j in range(1, k):
        m = jnp.maximum(m, ce[j:j + lcol, :])
    r = m[0:rtot, :]
    for i in range(1, k):
        r = jnp.maximum(r, m[i * wp: i * wp + rtot, :])
    return jnp.maximum(r, 0.0)


def _sel_odd_rows(m, b, hp, wp, c):
    q = m.reshape(b * hp * wp // 2, 2, c)[:, 1:2, :]
    q = q.reshape(b * hp // 2, wp, c)[:, wp // 2:, :]
    return q.reshape(b, hp // 2, wp // 2, c)


def _embed(sel, b, nq, wpo, c):
    zw = jnp.zeros((b, nq, 1, c), sel.dtype)
    row = jnp.concatenate([zw, sel, zw], axis=2)
    zh = jnp.zeros((b, 1, wpo, c), sel.dtype)
    out = jnp.concatenate([zh, row, zh], axis=1)
    return out.reshape(b * (nq + 2) * wpo, c)


def _route_pool(m, b, hp, wp, nq, hpo, wpo, c):
    sel = _sel_odd_rows(m, b, hp, wp, c)[:, :nq, :nq, :]
    return _embed(sel, b, nq, wpo, c)


def _zero_ring(cf, b, hp, wp, c):
    x4 = cf.reshape(b, hp, wp, c)
    inner = x4[:, 1:hp - 1, 1:wp - 1, :]
    zw = jnp.zeros((b, hp - 2, 1, c), cf.dtype)
    row = jnp.concatenate([zw, inner, zw], axis=2)
    zh = jnp.zeros((b, 1, wp, c), cf.dtype)
    return jnp.concatenate([zh, row, zh], axis=1).reshape(b * hp * wp, c)


def _net_kernel(x1_ref, w1_ref, b1_ref, w2_ref, b2_ref, w3_ref, b3_ref,
                w4_ref, b4_ref, w5_ref, b5_ref,
                f1_ref, fb1_ref, f2_ref, fb2_ref, f3_ref, fb3_ref, o_ref):
    b = _B
    x1 = x1_ref[...].reshape(b * _R1, 9)
    c1 = jnp.dot(x1, w1_ref[...], preferred_element_type=jnp.float32)
    c1 = c1 + b1_ref[...]
    a2 = _route_pool(_pool_relu(c1, _WP1, 2), b, _HP1, _WP1, 14,
                     _HP2, _WP2, 32)

    c2 = _conv3x3(a2, _WP2, w2_ref, b2_ref)
    a3 = _route_pool(_pool_relu(c2, _WP2, 2), b, _HP2, _WP2, 7,
                     _HP3, _WP3, 64)

    c3 = _conv3x3(a3, _WP3, w3_ref, b3_ref)
    a4 = _zero_ring(c3, b, _HP3, _WP3, 128)
    c4 = _conv3x3(a4, _WP3, w4_ref, b4_ref)
    a5 = _zero_ring(c4, b, _HP3, _WP3, 256)

    c5 = _conv3x3(a5, _WP3, w5_ref, b5_ref)
    m5 = _pool_relu(c5, _WP3, 3).reshape(b, _HP3, _WP3, 256)
    q = m5[:, :, 1:7, :].reshape(b * _HP3 * 3, 2, 256)[:, 0:1, :]
    q = q.reshape(b, _HP3, 3, 256)[:, 1:7, :, :]
    afc = q.reshape(b * 3, 6, 256)[:, 0:3, :].reshape(b, 3, 3, 256)

    h = fb1_ref[...]
    for i in range(3):
        for j in range(3):
            v = afc[:, i:i + 1, j:j + 1, :].reshape(b, 256).astype(jnp.bfloat16)
            h = h + jnp.dot(v, f1_ref[i * 3 + j],
                            preferred_element_type=jnp.float32)
    h = jnp.dot(h.astype(jnp.bfloat16), f2_ref[...],
                preferred_element_type=jnp.float32) + fb2_ref[...]
    out = jnp.dot(h.astype(jnp.bfloat16), f3_ref[...],
                  preferred_element_type=jnp.float32) + fb3_ref[...]
    o_ref[...] = out


def _conv1_taps(x_nchw):
    n = x_nchw.shape[0]
    x = x_nchw[:, 0, :, :].astype(jnp.float32)
    xp = jnp.pad(x, ((0, 0), (2, 2), (2, 2)))
    taps = [xp[:, dy:dy + _HP1, dx:dx + _WP1]
            for dy in range(3) for dx in range(3)]
    t = jnp.stack(taps, axis=-1)
    return t.reshape(n, _R1, 9).astype(jnp.bfloat16)


def _const_spec(a):
    nd = a.ndim
    return pl.BlockSpec(a.shape, lambda *_: (0,) * nd)


@jax.jit
def _forward(x, w1, b1, w2, b2, w3, b3, w4, b4, w5, b5,
             f1, fb1, f2, fb2, f3, fb3):
    n = x.shape[0]
    x1 = _conv1_taps(x)
    params = (w1, b1, w2, b2, w3, b3, w4, b4, w5, b5,
              f1, fb1, f2, fb2, f3, fb3)

    in_specs = [pl.BlockSpec((_B, _R1, 9), lambda g: (g, 0, 0))]
    in_specs += [_const_spec(p) for p in params]

    cost = pl.CostEstimate(
        flops=172_000_000 * n,
        transcendentals=0,
        bytes_accessed=9_000_000 + 2 * x1.size,
    )

    out = pl.pallas_call(
        _net_kernel,
        out_shape=jax.ShapeDtypeStruct((n, 10), jnp.float32),
        grid_spec=pltpu.PrefetchScalarGridSpec(
            num_scalar_prefetch=0,
            grid=(n // _B,),
            in_specs=in_specs,
            out_specs=pl.BlockSpec((_B, 10), lambda g: (g, 0)),
        ),
        compiler_params=pltpu.CompilerParams(
            dimension_semantics=("parallel",),
            vmem_limit_bytes=96 * 1024 * 1024,
        ),
        cost_estimate=cost,
    )(x1, *params)
    return out


def kernel(x, w1, b1, s1, w2, b2, s2, w3, b3, r34, w4, b4, w5, b5, s5,
           f1, fb1, f2, fb2, f3, fb3):
    del s1, s2, r34, s5
    return _forward(x, w1, b1, w2, b2, w3, b3, w4, b4, w5, b5,
                    f1, fb1, f2, fb2, f3, fb3)
